# baseline (speedup 1.0000x reference)
import functools

import jax
import jax.numpy as jnp
from jax import lax
from jax.experimental import pallas as pl
from jax.experimental.pallas import tpu as pltpu
from jax.experimental.pallas import tpu_sc as plsc

_B, _T, _D, _P = 16, 4096, 256, 3
_L = 16
_NC = 2
_DH = _D // 2
_NV = _DH // _L
_CH = 256
_S = 768
_BT = 256
_OP = 8

_mesh = plsc.VectorSubcoreMesh(core_axis_name="c", subcore_axis_name="s")


def _row_loop(buf, lo, hi, acc):
    @plsc.parallel_loop(lo, hi, carry=acc, unroll=4)
    def body(t, a):
        return tuple(
            jnp.maximum(a[j], buf[t, pl.ds(j * _L, _L)]) for j in range(_NV)
        )

    return body


def _compute_chunk(buf, t0, p0, p1, p2, accs):
    r0 = jnp.clip(p0 - t0, 0, _CH)
    r1 = jnp.clip(p1 - t0, 0, _CH)
    r2 = jnp.clip(p2 - t0, 0, _CH)
    a0, a1, a2 = accs
    a0 = _row_loop(buf, 0, r0, a0)
    a1 = _row_loop(buf, r0, r1, a1)
    a2 = _row_loop(buf, r1, r2, a2)
    return (a0, a1, a2)


@functools.partial(
    pl.kernel,
    out_type=jax.ShapeDtypeStruct((_B, _P, _D), jnp.float32),
    mesh=_mesh,
    scratch_types=[
        pltpu.VMEM((_L,), jnp.int32),
        pltpu.VMEM((_CH, _DH), jnp.float32),
        pltpu.VMEM((_CH, _DH), jnp.float32),
        pltpu.VMEM((_P, _DH), jnp.float32),
        pltpu.SemaphoreType.DMA,
        pltpu.SemaphoreType.DMA,
    ],
)
def _sc_pool(x_hbm, pos_hbm, out_hbm, pos_v, buf0, buf1, stage_v, sem0, sem1):
    c = lax.axis_index("c")
    s = lax.axis_index("s")
    wid = s * _NC + c
    b = wid // 2
    hoff = (wid % 2) * _DH

    pltpu.sync_copy(pos_hbm.at[b], pos_v)
    pvec = pos_v[...]
    p0 = pvec[0]
    p1 = pvec[1]
    p2 = pvec[2]

    neg = jnp.full((_L,), -jnp.inf, jnp.float32)
    acc_init = (tuple(neg for _ in range(_NV)),) * _P

    base = _S // _CH
    nch = jnp.maximum((p2 + _CH - 1) // _CH - base, 0)
    npair = (nch + 1) // 2

    def src(ci):
        return x_hbm.at[b, pl.ds((base + ci) * _CH, _CH), pl.ds(hoff, _DH)]

    @pl.when(nch > 0)
    def _():
        pltpu.async_copy(src(0), buf0, sem0)

    def body(k, accs):
        ci0 = 2 * k
        ci1 = ci0 + 1

        pltpu.make_async_copy(src(ci0), buf0, sem0).wait()

        @pl.when(ci1 < nch)
        def _():
            pltpu.async_copy(src(ci1), buf1, sem1)

        accs = _compute_chunk(buf0, (base + ci0) * _CH, p0, p1, p2, accs)

        @pl.when(ci0 + 2 < nch)
        def _():
            pltpu.async_copy(src(ci0 + 2), buf0, sem0)

        @pl.when(ci1 < nch)
        def _():
            pltpu.make_async_copy(src(ci1), buf1, sem1).wait()

        accs = _compute_chunk(buf1, (base + ci1) * _CH, p0, p1, p2, accs)
        return accs

    accs = lax.fori_loop(0, npair, body, acc_init)

    for p in range(_P):
        for j in range(_NV):
            stage_v[p, pl.ds(j * _L, _L)] = accs[p][j]
    pltpu.sync_copy(stage_v, out_hbm.at[b, :, pl.ds(hoff, _DH)])


def _tc_body(pos_ref, x_ref, o_ref):
    b = pl.program_id(0)
    i = pl.program_id(1)

    @pl.when(i == 0)
    def _init():
        o_ref[...] = jnp.full(o_ref.shape, -jnp.inf, o_ref.dtype)

    p0 = pos_ref[b, 0]
    p1 = pos_ref[b, 1]
    p2 = pos_ref[b, 2]
    t = i * _BT + jax.lax.broadcasted_iota(jnp.int32, (_BT, 1), 0)
    x = x_ref[0]
    seg = (
        (t >= p0).astype(jnp.int32)
        + (t >= p1).astype(jnp.int32)
        + (t >= p2).astype(jnp.int32)
    )
    rows = []
    for p in range(_P):
        rows.append(
            jnp.max(jnp.where(seg == p, x, -jnp.inf), axis=0, keepdims=True)
        )
    new = jnp.concatenate(rows, axis=0)
    o_ref[0, :_P, :] = jnp.maximum(o_ref[0, :_P, :], new)


def _tc_head(inputs, positions):
    grid_spec = pltpu.PrefetchScalarGridSpec(
        num_scalar_prefetch=1,
        grid=(_B, _S // _BT),
        in_specs=[pl.BlockSpec((1, _BT, _D), lambda b, i, pos: (b, i, 0))],
        out_specs=pl.BlockSpec((1, _OP, _D), lambda b, i, pos: (b, 0, 0)),
    )
    return pl.pallas_call(
        _tc_body,
        grid_spec=grid_spec,
        out_shape=jax.ShapeDtypeStruct((_B, _OP, _D), jnp.float32),
    )(positions, inputs)


def _combine_body(tc_ref, sc_ref, o_ref):
    o_ref[...] = jnp.maximum(tc_ref[:, :_P, :], sc_ref[...])


def _combine(tc_part, sc_part):
    return pl.pallas_call(
        _combine_body,
        out_shape=jax.ShapeDtypeStruct((_B, _P, _D), jnp.float32),
    )(tc_part, sc_part)


def kernel(inputs, positions):
    pos_pad = jnp.zeros((_B, _L), jnp.int32).at[:, :_P].set(positions)
    sc_part = _sc_pool(inputs, pos_pad)
    tc_part = _tc_head(inputs, positions)
    return _combine(tc_part, sc_part)


# speedup vs baseline: 1.1415x; 1.1415x over previous
import functools

import jax
import jax.numpy as jnp
from jax import lax
from jax.experimental import pallas as pl
from jax.experimental.pallas import tpu as pltpu
from jax.experimental.pallas import tpu_sc as plsc

_B, _T, _D, _P = 16, 4096, 256, 3
_L = 16
_NC = 2
_DH = _D // 2
_NV = _DH // _L
_CH = 256
_S = 1024
_BT = 512
_OP = 8

_mesh = plsc.VectorSubcoreMesh(core_axis_name="c", subcore_axis_name="s")


def _row_loop(buf, lo, hi, acc):
    @plsc.parallel_loop(lo, hi, carry=acc, unroll=4)
    def body(t, a):
        return tuple(
            jnp.maximum(a[j], buf[t, pl.ds(j * _L, _L)]) for j in range(_NV)
        )

    return body


def _compute_chunk(buf, t0, p0, p1, p2, accs):
    r0 = jnp.clip(p0 - t0, 0, _CH)
    r1 = jnp.clip(p1 - t0, 0, _CH)
    r2 = jnp.clip(p2 - t0, 0, _CH)
    a0, a1, a2 = accs
    a0 = _row_loop(buf, 0, r0, a0)
    a1 = _row_loop(buf, r0, r1, a1)
    a2 = _row_loop(buf, r1, r2, a2)
    return (a0, a1, a2)


@functools.partial(
    pl.kernel,
    out_type=jax.ShapeDtypeStruct((_B, _P, _D), jnp.float32),
    mesh=_mesh,
    scratch_types=[
        pltpu.VMEM((_L,), jnp.int32),
        pltpu.VMEM((_CH, _DH), jnp.float32),
        pltpu.VMEM((_CH, _DH), jnp.float32),
        pltpu.VMEM((_P, _DH), jnp.float32),
        pltpu.SemaphoreType.DMA,
        pltpu.SemaphoreType.DMA,
    ],
)
def _sc_pool(x_hbm, pos_hbm, out_hbm, pos_v, buf0, buf1, stage_v, sem0, sem1):
    c = lax.axis_index("c")
    s = lax.axis_index("s")
    wid = s * _NC + c
    b = wid // 2
    hoff = (wid % 2) * _DH

    pltpu.sync_copy(pos_hbm.at[b], pos_v)
    pvec = pos_v[...]
    p0 = pvec[0]
    p1 = pvec[1]
    p2 = pvec[2]

    neg = jnp.full((_L,), -jnp.inf, jnp.float32)
    acc_init = (tuple(neg for _ in range(_NV)),) * _P

    base = _S // _CH
    nch = jnp.maximum((p2 + _CH - 1) // _CH - base, 0)
    npair = (nch + 1) // 2

    def src(ci):
        return x_hbm.at[b, pl.ds((base + ci) * _CH, _CH), pl.ds(hoff, _DH)]

    @pl.when(nch > 0)
    def _():
        pltpu.async_copy(src(0), buf0, sem0)

    def body(k, accs):
        ci0 = 2 * k
        ci1 = ci0 + 1

        pltpu.make_async_copy(src(ci0), buf0, sem0).wait()

        @pl.when(ci1 < nch)
        def _():
            pltpu.async_copy(src(ci1), buf1, sem1)

        accs = _compute_chunk(buf0, (base + ci0) * _CH, p0, p1, p2, accs)

        @pl.when(ci0 + 2 < nch)
        def _():
            pltpu.async_copy(src(ci0 + 2), buf0, sem0)

        @pl.when(ci1 < nch)
        def _():
            pltpu.make_async_copy(src(ci1), buf1, sem1).wait()

        accs = _compute_chunk(buf1, (base + ci1) * _CH, p0, p1, p2, accs)
        return accs

    accs = lax.fori_loop(0, npair, body, acc_init)

    for p in range(_P):
        for j in range(_NV):
            stage_v[p, pl.ds(j * _L, _L)] = accs[p][j]
    pltpu.sync_copy(stage_v, out_hbm.at[b, :, pl.ds(hoff, _DH)])


def _tc_body(pos_ref, x_ref, o_ref):
    b = pl.program_id(0)
    i = pl.program_id(1)

    @pl.when(i == 0)
    def _init():
        o_ref[...] = jnp.full(o_ref.shape, -jnp.inf, o_ref.dtype)

    p0 = pos_ref[b, 0]
    p1 = pos_ref[b, 1]
    p2 = pos_ref[b, 2]
    t = i * _BT + jax.lax.broadcasted_iota(jnp.int32, (_BT, 1), 0)
    x = x_ref[0]
    seg = (
        (t >= p0).astype(jnp.int32)
        + (t >= p1).astype(jnp.int32)
        + (t >= p2).astype(jnp.int32)
    )
    rows = []
    for p in range(_P):
        rows.append(
            jnp.max(jnp.where(seg == p, x, -jnp.inf), axis=0, keepdims=True)
        )
    new = jnp.concatenate(rows, axis=0)
    o_ref[0, :_P, :] = jnp.maximum(o_ref[0, :_P, :], new)


def _tc_head(inputs, positions):
    grid_spec = pltpu.PrefetchScalarGridSpec(
        num_scalar_prefetch=1,
        grid=(_B, _S // _BT),
        in_specs=[pl.BlockSpec((1, _BT, _D), lambda b, i, pos: (b, i, 0))],
        out_specs=pl.BlockSpec((1, _OP, _D), lambda b, i, pos: (b, 0, 0)),
    )
    return pl.pallas_call(
        _tc_body,
        grid_spec=grid_spec,
        out_shape=jax.ShapeDtypeStruct((_B, _OP, _D), jnp.float32),
    )(positions, inputs)


def _combine_body(tc_ref, sc_ref, o_ref):
    o_ref[...] = jnp.maximum(tc_ref[:, :_P, :], sc_ref[...])


def _combine(tc_part, sc_part):
    return pl.pallas_call(
        _combine_body,
        out_shape=jax.ShapeDtypeStruct((_B, _P, _D), jnp.float32),
    )(tc_part, sc_part)


def kernel(inputs, positions):
    pos_pad = jnp.pad(positions, ((0, 0), (0, _L - _P)))
    sc_part = _sc_pool(inputs, pos_pad)
    tc_part = _tc_head(inputs, positions)
    return _combine(tc_part, sc_part)


# speedup vs baseline: 1.1468x; 1.0047x over previous
import functools

import jax
import jax.numpy as jnp
from jax import lax
from jax.experimental import pallas as pl
from jax.experimental.pallas import tpu as pltpu
from jax.experimental.pallas import tpu_sc as plsc

_B, _T, _D, _P = 16, 4096, 256, 3
_L = 16
_NC = 2
_DH = _D // 2
_NV = _DH // _L
_CH = 256
_S = 1024
_BT = 1024
_OP = 8

_mesh = plsc.VectorSubcoreMesh(core_axis_name="c", subcore_axis_name="s")


def _row_loop(buf, lo, hi, acc):
    @plsc.parallel_loop(lo, hi, carry=acc, unroll=4)
    def body(t, a):
        return tuple(
            jnp.maximum(a[j], buf[t, pl.ds(j * _L, _L)]) for j in range(_NV)
        )

    return body


def _compute_chunk(buf, t0, p0, p1, p2, accs):
    r0 = jnp.clip(p0 - t0, 0, _CH)
    r1 = jnp.clip(p1 - t0, 0, _CH)
    r2 = jnp.clip(p2 - t0, 0, _CH)
    a0, a1, a2 = accs
    a0 = _row_loop(buf, 0, r0, a0)
    a1 = _row_loop(buf, r0, r1, a1)
    a2 = _row_loop(buf, r1, r2, a2)
    return (a0, a1, a2)


@functools.partial(
    pl.kernel,
    out_type=jax.ShapeDtypeStruct((_B, _P, _D), jnp.float32),
    mesh=_mesh,
    scratch_types=[
        pltpu.VMEM((_L,), jnp.int32),
        pltpu.VMEM((_CH, _DH), jnp.float32),
        pltpu.VMEM((_CH, _DH), jnp.float32),
        pltpu.VMEM((_P, _DH), jnp.float32),
        pltpu.SemaphoreType.DMA,
        pltpu.SemaphoreType.DMA,
    ],
)
def _sc_pool(x_hbm, pos_hbm, out_hbm, pos_v, buf0, buf1, stage_v, sem0, sem1):
    c = lax.axis_index("c")
    s = lax.axis_index("s")
    wid = s * _NC + c
    b = wid // 2
    hoff = (wid % 2) * _DH

    pltpu.sync_copy(pos_hbm.at[b], pos_v)
    pvec = pos_v[...]
    p0 = pvec[0]
    p1 = pvec[1]
    p2 = pvec[2]

    neg = jnp.full((_L,), -jnp.inf, jnp.float32)
    acc_init = (tuple(neg for _ in range(_NV)),) * _P

    base = _S // _CH
    nch = jnp.maximum((p2 + _CH - 1) // _CH - base, 0)
    npair = (nch + 1) // 2

    def src(ci):
        return x_hbm.at[b, pl.ds((base + ci) * _CH, _CH), pl.ds(hoff, _DH)]

    @pl.when(nch > 0)
    def _():
        pltpu.async_copy(src(0), buf0, sem0)

    def body(k, accs):
        ci0 = 2 * k
        ci1 = ci0 + 1

        pltpu.make_async_copy(src(ci0), buf0, sem0).wait()

        @pl.when(ci1 < nch)
        def _():
            pltpu.async_copy(src(ci1), buf1, sem1)

        accs = _compute_chunk(buf0, (base + ci0) * _CH, p0, p1, p2, accs)

        @pl.when(ci0 + 2 < nch)
        def _():
            pltpu.async_copy(src(ci0 + 2), buf0, sem0)

        @pl.when(ci1 < nch)
        def _():
            pltpu.make_async_copy(src(ci1), buf1, sem1).wait()

        accs = _compute_chunk(buf1, (base + ci1) * _CH, p0, p1, p2, accs)
        return accs

    accs = lax.fori_loop(0, npair, body, acc_init)

    for p in range(_P):
        for j in range(_NV):
            stage_v[p, pl.ds(j * _L, _L)] = accs[p][j]
    pltpu.sync_copy(stage_v, out_hbm.at[b, :, pl.ds(hoff, _DH)])


def _tc_body(pos_ref, x_ref, o_ref):
    b = pl.program_id(0)
    i = pl.program_id(1)

    @pl.when(i == 0)
    def _init():
        o_ref[...] = jnp.full(o_ref.shape, -jnp.inf, o_ref.dtype)

    p0 = pos_ref[b, 0]
    p1 = pos_ref[b, 1]
    p2 = pos_ref[b, 2]
    t = i * _BT + jax.lax.broadcasted_iota(jnp.int32, (_BT, 1), 0)
    x = x_ref[0]
    seg = (
        (t >= p0).astype(jnp.int32)
        + (t >= p1).astype(jnp.int32)
        + (t >= p2).astype(jnp.int32)
    )
    rows = []
    for p in range(_P):
        rows.append(
            jnp.max(jnp.where(seg == p, x, -jnp.inf), axis=0, keepdims=True)
        )
    new = jnp.concatenate(rows, axis=0)
    o_ref[0, :_P, :] = jnp.maximum(o_ref[0, :_P, :], new)


def _tc_head(inputs, positions):
    grid_spec = pltpu.PrefetchScalarGridSpec(
        num_scalar_prefetch=1,
        grid=(_B, _S // _BT),
        in_specs=[pl.BlockSpec((1, _BT, _D), lambda b, i, pos: (b, i, 0))],
        out_specs=pl.BlockSpec((1, _OP, _D), lambda b, i, pos: (b, 0, 0)),
    )
    return pl.pallas_call(
        _tc_body,
        grid_spec=grid_spec,
        out_shape=jax.ShapeDtypeStruct((_B, _OP, _D), jnp.float32),
    )(positions, inputs)


def _combine_body(tc_ref, sc_ref, o_ref):
    o_ref[...] = jnp.maximum(tc_ref[:, :_P, :], sc_ref[...])


def _combine(tc_part, sc_part):
    return pl.pallas_call(
        _combine_body,
        out_shape=jax.ShapeDtypeStruct((_B, _P, _D), jnp.float32),
    )(tc_part, sc_part)


def kernel(inputs, positions):
    pos_pad = jnp.pad(positions, ((0, 0), (0, _L - _P)))
    sc_part = _sc_pool(inputs, pos_pad)
    tc_part = _tc_head(inputs, positions)
    return _combine(tc_part, sc_part)


# speedup vs baseline: 1.1669x; 1.0175x over previous
import functools

import jax
import jax.numpy as jnp
from jax import lax
from jax.experimental import pallas as pl
from jax.experimental.pallas import tpu as pltpu
from jax.experimental.pallas import tpu_sc as plsc

_B, _T, _D, _P = 16, 4096, 256, 3
_L = 16
_NC = 2
_DH = _D // 2
_NV = _DH // _L
_CH = 256
_S = 1280
_BT = 1280
_OP = 8

_mesh = plsc.VectorSubcoreMesh(core_axis_name="c", subcore_axis_name="s")


def _row_loop(buf, lo, hi, acc):
    @plsc.parallel_loop(lo, hi, carry=acc, unroll=4)
    def body(t, a):
        return tuple(
            jnp.maximum(a[j], buf[t, pl.ds(j * _L, _L)]) for j in range(_NV)
        )

    return body


def _compute_chunk(buf, t0, p0, p1, p2, accs):
    r0 = jnp.clip(p0 - t0, 0, _CH)
    r1 = jnp.clip(p1 - t0, 0, _CH)
    r2 = jnp.clip(p2 - t0, 0, _CH)
    a0, a1, a2 = accs
    a0 = _row_loop(buf, 0, r0, a0)
    a1 = _row_loop(buf, r0, r1, a1)
    a2 = _row_loop(buf, r1, r2, a2)
    return (a0, a1, a2)


@functools.partial(
    pl.kernel,
    out_type=jax.ShapeDtypeStruct((_B, _P, _D), jnp.float32),
    mesh=_mesh,
    scratch_types=[
        pltpu.VMEM((_L,), jnp.int32),
        pltpu.VMEM((_CH, _DH), jnp.float32),
        pltpu.VMEM((_CH, _DH), jnp.float32),
        pltpu.VMEM((_P, _DH), jnp.float32),
        pltpu.SemaphoreType.DMA,
        pltpu.SemaphoreType.DMA,
    ],
)
def _sc_pool(x_hbm, pos_hbm, out_hbm, pos_v, buf0, buf1, stage_v, sem0, sem1):
    c = lax.axis_index("c")
    s = lax.axis_index("s")
    wid = s * _NC + c
    b = wid // 2
    hoff = (wid % 2) * _DH

    pltpu.sync_copy(pos_hbm.at[b], pos_v)
    pvec = pos_v[...]
    p0 = pvec[0]
    p1 = pvec[1]
    p2 = pvec[2]

    neg = jnp.full((_L,), -jnp.inf, jnp.float32)
    acc_init = (tuple(neg for _ in range(_NV)),) * _P

    base = _S // _CH
    nch = jnp.maximum((p2 + _CH - 1) // _CH - base, 0)
    npair = (nch + 1) // 2

    def src(ci):
        return x_hbm.at[b, pl.ds((base + ci) * _CH, _CH), pl.ds(hoff, _DH)]

    @pl.when(nch > 0)
    def _():
        pltpu.async_copy(src(0), buf0, sem0)

    def body(k, accs):
        ci0 = 2 * k
        ci1 = ci0 + 1

        pltpu.make_async_copy(src(ci0), buf0, sem0).wait()

        @pl.when(ci1 < nch)
        def _():
            pltpu.async_copy(src(ci1), buf1, sem1)

        accs = _compute_chunk(buf0, (base + ci0) * _CH, p0, p1, p2, accs)

        @pl.when(ci0 + 2 < nch)
        def _():
            pltpu.async_copy(src(ci0 + 2), buf0, sem0)

        @pl.when(ci1 < nch)
        def _():
            pltpu.make_async_copy(src(ci1), buf1, sem1).wait()

        accs = _compute_chunk(buf1, (base + ci1) * _CH, p0, p1, p2, accs)
        return accs

    accs = lax.fori_loop(0, npair, body, acc_init)

    for p in range(_P):
        for j in range(_NV):
            stage_v[p, pl.ds(j * _L, _L)] = accs[p][j]
    pltpu.sync_copy(stage_v, out_hbm.at[b, :, pl.ds(hoff, _DH)])


def _tc_body(pos_ref, x_ref, o_ref):
    b = pl.program_id(0)
    i = pl.program_id(1)

    @pl.when(i == 0)
    def _init():
        o_ref[...] = jnp.full(o_ref.shape, -jnp.inf, o_ref.dtype)

    p0 = pos_ref[b, 0]
    p1 = pos_ref[b, 1]
    p2 = pos_ref[b, 2]
    t = i * _BT + jax.lax.broadcasted_iota(jnp.int32, (_BT, 1), 0)
    x = x_ref[0]
    seg = (
        (t >= p0).astype(jnp.int32)
        + (t >= p1).astype(jnp.int32)
        + (t >= p2).astype(jnp.int32)
    )
    rows = []
    for p in range(_P):
        rows.append(
            jnp.max(jnp.where(seg == p, x, -jnp.inf), axis=0, keepdims=True)
        )
    new = jnp.concatenate(rows, axis=0)
    o_ref[0, :_P, :] = jnp.maximum(o_ref[0, :_P, :], new)


def _tc_head(inputs, positions):
    grid_spec = pltpu.PrefetchScalarGridSpec(
        num_scalar_prefetch=1,
        grid=(_B, _S // _BT),
        in_specs=[pl.BlockSpec((1, _BT, _D), lambda b, i, pos: (b, i, 0))],
        out_specs=pl.BlockSpec((1, _OP, _D), lambda b, i, pos: (b, 0, 0)),
    )
    return pl.pallas_call(
        _tc_body,
        grid_spec=grid_spec,
        out_shape=jax.ShapeDtypeStruct((_B, _OP, _D), jnp.float32),
    )(positions, inputs)


def _combine_body(tc_ref, sc_ref, o_ref):
    o_ref[...] = jnp.maximum(tc_ref[:, :_P, :], sc_ref[...])


def _combine(tc_part, sc_part):
    return pl.pallas_call(
        _combine_body,
        out_shape=jax.ShapeDtypeStruct((_B, _P, _D), jnp.float32),
    )(tc_part, sc_part)


def kernel(inputs, positions):
    pos_pad = jnp.pad(positions, ((0, 0), (0, _L - _P)))
    sc_part = _sc_pool(inputs, pos_pad)
    tc_part = _tc_head(inputs, positions)
    return _combine(tc_part, sc_part)


# speedup vs baseline: 1.1939x; 1.0231x over previous
import functools

import jax
import jax.numpy as jnp
from jax import lax
from jax.experimental import pallas as pl
from jax.experimental.pallas import tpu as pltpu
from jax.experimental.pallas import tpu_sc as plsc

_B, _T, _D, _P = 16, 4096, 256, 3
_L = 16
_NC = 2
_DH = _D // 2
_NV = _DH // _L
_CH = 256
_S = 1536
_BT = 1536
_OP = 8

_mesh = plsc.VectorSubcoreMesh(core_axis_name="c", subcore_axis_name="s")


def _row_loop(buf, lo, hi, acc):
    @plsc.parallel_loop(lo, hi, carry=acc, unroll=4)
    def body(t, a):
        return tuple(
            jnp.maximum(a[j], buf[t, pl.ds(j * _L, _L)]) for j in range(_NV)
        )

    return body


def _compute_chunk(buf, t0, p0, p1, p2, accs):
    r0 = jnp.clip(p0 - t0, 0, _CH)
    r1 = jnp.clip(p1 - t0, 0, _CH)
    r2 = jnp.clip(p2 - t0, 0, _CH)
    a0, a1, a2 = accs
    a0 = _row_loop(buf, 0, r0, a0)
    a1 = _row_loop(buf, r0, r1, a1)
    a2 = _row_loop(buf, r1, r2, a2)
    return (a0, a1, a2)


@functools.partial(
    pl.kernel,
    out_type=jax.ShapeDtypeStruct((_B, _P, _D), jnp.float32),
    mesh=_mesh,
    scratch_types=[
        pltpu.VMEM((_L,), jnp.int32),
        pltpu.VMEM((_CH, _DH), jnp.float32),
        pltpu.VMEM((_CH, _DH), jnp.float32),
        pltpu.VMEM((_P, _DH), jnp.float32),
        pltpu.SemaphoreType.DMA,
        pltpu.SemaphoreType.DMA,
    ],
)
def _sc_pool(x_hbm, pos_hbm, out_hbm, pos_v, buf0, buf1, stage_v, sem0, sem1):
    c = lax.axis_index("c")
    s = lax.axis_index("s")
    wid = s * _NC + c
    b = wid // 2
    hoff = (wid % 2) * _DH

    pltpu.sync_copy(pos_hbm.at[b], pos_v)
    pvec = pos_v[...]
    p0 = pvec[0]
    p1 = pvec[1]
    p2 = pvec[2]

    neg = jnp.full((_L,), -jnp.inf, jnp.float32)
    acc_init = (tuple(neg for _ in range(_NV)),) * _P

    base = _S // _CH
    nch = jnp.maximum((p2 + _CH - 1) // _CH - base, 0)
    npair = (nch + 1) // 2

    def src(ci):
        return x_hbm.at[b, pl.ds((base + ci) * _CH, _CH), pl.ds(hoff, _DH)]

    @pl.when(nch > 0)
    def _():
        pltpu.async_copy(src(0), buf0, sem0)

    def body(k, accs):
        ci0 = 2 * k
        ci1 = ci0 + 1

        pltpu.make_async_copy(src(ci0), buf0, sem0).wait()

        @pl.when(ci1 < nch)
        def _():
            pltpu.async_copy(src(ci1), buf1, sem1)

        accs = _compute_chunk(buf0, (base + ci0) * _CH, p0, p1, p2, accs)

        @pl.when(ci0 + 2 < nch)
        def _():
            pltpu.async_copy(src(ci0 + 2), buf0, sem0)

        @pl.when(ci1 < nch)
        def _():
            pltpu.make_async_copy(src(ci1), buf1, sem1).wait()

        accs = _compute_chunk(buf1, (base + ci1) * _CH, p0, p1, p2, accs)
        return accs

    accs = lax.fori_loop(0, npair, body, acc_init)

    for p in range(_P):
        for j in range(_NV):
            stage_v[p, pl.ds(j * _L, _L)] = accs[p][j]
    pltpu.sync_copy(stage_v, out_hbm.at[b, :, pl.ds(hoff, _DH)])


def _tc_body(pos_ref, x_ref, o_ref):
    b = pl.program_id(0)
    i = pl.program_id(1)

    @pl.when(i == 0)
    def _init():
        o_ref[...] = jnp.full(o_ref.shape, -jnp.inf, o_ref.dtype)

    p0 = pos_ref[b, 0]
    p1 = pos_ref[b, 1]
    p2 = pos_ref[b, 2]
    t = i * _BT + jax.lax.broadcasted_iota(jnp.int32, (_BT, 1), 0)
    x = x_ref[0]
    seg = (
        (t >= p0).astype(jnp.int32)
        + (t >= p1).astype(jnp.int32)
        + (t >= p2).astype(jnp.int32)
    )
    rows = []
    for p in range(_P):
        rows.append(
            jnp.max(jnp.where(seg == p, x, -jnp.inf), axis=0, keepdims=True)
        )
    new = jnp.concatenate(rows, axis=0)
    o_ref[0, :_P, :] = jnp.maximum(o_ref[0, :_P, :], new)


def _tc_head(inputs, positions):
    grid_spec = pltpu.PrefetchScalarGridSpec(
        num_scalar_prefetch=1,
        grid=(_B, _S // _BT),
        in_specs=[pl.BlockSpec((1, _BT, _D), lambda b, i, pos: (b, i, 0))],
        out_specs=pl.BlockSpec((1, _OP, _D), lambda b, i, pos: (b, 0, 0)),
    )
    return pl.pallas_call(
        _tc_body,
        grid_spec=grid_spec,
        out_shape=jax.ShapeDtypeStruct((_B, _OP, _D), jnp.float32),
    )(positions, inputs)


def _combine_body(tc_ref, sc_ref, o_ref):
    o_ref[...] = jnp.maximum(tc_ref[:, :_P, :], sc_ref[...])


def _combine(tc_part, sc_part):
    return pl.pallas_call(
        _combine_body,
        out_shape=jax.ShapeDtypeStruct((_B, _P, _D), jnp.float32),
    )(tc_part, sc_part)


def kernel(inputs, positions):
    pos_pad = jnp.pad(positions, ((0, 0), (0, _L - _P)))
    sc_part = _sc_pool(inputs, pos_pad)
    tc_part = _tc_head(inputs, positions)
    return _combine(tc_part, sc_part)
